# baseline (speedup 1.0000x reference)
import dataclasses
import functools

import jax
import jax.numpy as jnp
from jax import lax
from jax.experimental import pallas as pl
from jax.experimental.pallas import tpu as pltpu
from jax.experimental.pallas import tpu_sc as plsc

E = 8
D = 1024
F = 768
T = 2048
P2 = 2 * T
BLKR = 256
NBLK = P2 // BLKR + E
NSLOT = NBLK * BLKR
TRASH = P2

NTILE = 16
PP = P2 // NTILE
NCH = PP // 16


def _router_body(x_ref, wg_ref, s1_ref, s2_ref, w1_ref, w2_ref):
    x = x_ref[...].astype(jnp.bfloat16)
    gw = wg_ref[...].astype(jnp.bfloat16)
    logits = jax.lax.dot_general(
        x, gw, (((1,), (1,)), ((), ())),
        preferred_element_type=jnp.float32,
    )
    m = jnp.max(logits, axis=1, keepdims=True)
    p = jnp.exp(logits - m)
    p = p / jnp.sum(p, axis=1, keepdims=True)
    lane = jax.lax.broadcasted_iota(jnp.int32, (T, E), 1)
    m1 = jnp.max(p, axis=1, keepdims=True)
    a1 = jnp.min(jnp.where(p == m1, lane, E), axis=1, keepdims=True)
    pm = jnp.where(lane == a1, -1.0, p)
    m2 = jnp.max(pm, axis=1, keepdims=True)
    a2 = jnp.min(jnp.where(pm == m2, lane, E), axis=1, keepdims=True)
    s = m1 + m2
    s1_ref[...] = a1
    s2_ref[...] = a2
    w1_ref[...] = m1 / s
    w2_ref[...] = m2 / s


def _dispatch_body(sel_hbm, w_hbm, be_hbm, dest_hbm, tok_hbm, wvec_hbm,
                   lc_hbm, sel_v, w_v, lrank_v, p_v, t_v, s_v0, s_v1,
                   run_v, lc_all, offs_v, be_v, zi_v, zc_v, zf_v):
    wid = lax.axis_index("s")
    lane16 = jax.lax.broadcasted_iota(jnp.int32, (16,), 0)
    zeros16 = jnp.zeros((16,), jnp.int32)

    share = NSLOT // NTILE
    for j in range(share // 16):
        sl = pl.ds(j * 16, 16)
        zi_v[sl] = zeros16
        zc_v[sl] = zeros16 + TRASH
        zf_v[sl] = jnp.zeros((16,), jnp.float32)
    base_s = wid * share
    pltpu.sync_copy(zc_v, dest_hbm.at[pl.ds(base_s, share)])
    pltpu.sync_copy(zi_v, tok_hbm.at[pl.ds(base_s, share)])
    pltpu.sync_copy(zf_v, wvec_hbm.at[pl.ds(base_s, share)])

    base_p = wid * PP
    pltpu.sync_copy(sel_hbm.at[pl.ds(base_p, PP)], sel_v)
    pltpu.sync_copy(w_hbm.at[pl.ds(base_p, PP)], w_v)
    run = zeros16
    for i in range(NCH):
        sl = pl.ds(i * 16, 16)
        v = sel_v[sl]
        lrank = zeros16
        for e in range(E):
            msk = v == e
            c = plsc.cumsum(jnp.where(msk, 1, 0))
            cnt_e = jnp.max(c)
            prev_e = jnp.sum(jnp.where(lane16 == e, run, 0))
            lrank = lrank + jnp.where(msk, prev_e + c - 1, 0)
            run = run + jnp.where(lane16 == e, cnt_e, 0)
        lrank_v[sl] = lrank
    run_v[...] = run
    pltpu.sync_copy(run_v, lc_hbm.at[pl.ds(wid * 16, 16)])
    plsc.subcore_barrier()

    pltpu.sync_copy(lc_hbm, lc_all)
    wid_v = zeros16 + wid
    pre = zeros16
    tot = zeros16
    for k2 in range(NTILE):
        row = lc_all[pl.ds(k2 * 16, 16)]
        tot = tot + row
        pre = pre + jnp.where(zeros16 + k2 < wid_v, row, 0)
    pe = ((tot + (BLKR - 1)) >> 8) << 8
    seg_incl = plsc.cumsum(pe)
    seg_off = seg_incl - pe
    offs_v[...] = seg_off + pre

    bstart = seg_off >> 8
    for j in range(2):
        bvec = lane16 + 16 * j
        acc = zeros16 - 1
        for e in range(E):
            bs_e = jnp.sum(jnp.where(lane16 == e, bstart, 0))
            acc = acc + jnp.where(bvec >= zeros16 + bs_e, 1, 0)
        be_v[pl.ds(16 * j, 16)] = acc

    @pl.when(wid == 0)
    def _():
        pltpu.sync_copy(be_v, be_hbm)

    for i in range(NCH):
        sl = pl.ds(i * 16, 16)
        v = sel_v[sl]
        ge = plsc.load_gather(offs_v, [v])
        s = ge + lrank_v[sl]
        if i < NCH // 2:
            s_v0[pl.ds(i * 16, 16)] = s
        else:
            s_v1[pl.ds((i - NCH // 2) * 16, 16)] = s
        pvals = zeros16 + (base_p + i * 16) + lane16
        p_v[sl] = pvals
        t_v[sl] = pvals & (T - 1)
    half = PP // 2
    pltpu.sync_copy(p_v.at[pl.ds(0, half)], dest_hbm.at[s_v0])
    pltpu.sync_copy(p_v.at[pl.ds(half, half)], dest_hbm.at[s_v1])
    pltpu.sync_copy(t_v.at[pl.ds(0, half)], tok_hbm.at[s_v0])
    pltpu.sync_copy(t_v.at[pl.ds(half, half)], tok_hbm.at[s_v1])
    pltpu.sync_copy(w_v.at[pl.ds(0, half)], wvec_hbm.at[s_v0])
    pltpu.sync_copy(w_v.at[pl.ds(half, half)], wvec_hbm.at[s_v1])


GW = 48


def _gather_kernel(x_hbm, tok_hbm, xg_hbm):
    def body(i_vmem, o_vmem):
        pltpu.sync_copy(x_hbm.at[i_vmem.at[0]], o_vmem)

    pltpu.emit_pipeline(
        body,
        grid=(NSLOT // GW,),
        in_specs=[pl.BlockSpec((1, GW), lambda i: (i, 0))],
        out_specs=[pl.BlockSpec((GW, D), lambda i: (i, 0))],
        core_axis_name=("c", "s"),
        dimension_semantics=(pltpu.PARALLEL,),
    )(tok_hbm, xg_hbm)


def _scatter_kernel(yw_hbm, dest_hbm, buf_hbm):
    def body(x_vmem, i_vmem):
        pltpu.sync_copy(x_vmem, buf_hbm.at[i_vmem.at[0]])

    pltpu.emit_pipeline(
        body,
        grid=(NSLOT // GW,),
        in_specs=[
            pl.BlockSpec((GW, D), lambda i: (i, 0)),
            pl.BlockSpec((1, GW), lambda i: (i, 0)),
        ],
        out_specs=[],
        core_axis_name=("c", "s"),
        dimension_semantics=(pltpu.PARALLEL,),
    )(yw_hbm, dest_hbm)


def _ffn_body(be_ref, xg_ref, wg_ref, wu_ref, wd_ref, wv_ref, yw_ref):
    xb = xg_ref[...].astype(jnp.bfloat16)
    g = jnp.dot(xb, wg_ref[0], preferred_element_type=jnp.float32)
    u = jnp.dot(xb, wu_ref[0], preferred_element_type=jnp.float32)
    h = (g * jax.nn.sigmoid(g)) * u
    y = jnp.dot(h.astype(jnp.bfloat16), wd_ref[0],
                preferred_element_type=jnp.float32)
    yw_ref[...] = y * wv_ref[...]


def _add_body(a_ref, b_ref, out_ref):
    out_ref[...] = a_ref[...] + b_ref[...]


@jax.jit
def kernel(hidden_states, W_gate, W_g, W_u, W_d):
    orig_shape = hidden_states.shape
    x = hidden_states.reshape(T, D)

    s1, s2, w1, w2 = pl.pallas_call(
        _router_body,
        out_shape=(
            jax.ShapeDtypeStruct((T, 1), jnp.int32),
            jax.ShapeDtypeStruct((T, 1), jnp.int32),
            jax.ShapeDtypeStruct((T, 1), jnp.float32),
            jax.ShapeDtypeStruct((T, 1), jnp.float32),
        ),
    )(x, W_gate)
    sel_all = jnp.concatenate([s1, s2], axis=0).reshape(P2)
    w_all = jnp.concatenate([w1, w2], axis=0).reshape(P2)

    cp = pltpu.CompilerParams()
    if "needs_layout_passes" in pltpu.CompilerParams.__dataclass_fields__:
        cp = dataclasses.replace(cp, needs_layout_passes=False)
    mesh1 = plsc.VectorSubcoreMesh(
        core_axis_name="c", subcore_axis_name="s", num_cores=1)
    dispatch = pl.kernel(
        _dispatch_body,
        out_type=(
            jax.ShapeDtypeStruct((2 * NTILE,), jnp.int32),
            jax.ShapeDtypeStruct((NSLOT,), jnp.int32),
            jax.ShapeDtypeStruct((NSLOT,), jnp.int32),
            jax.ShapeDtypeStruct((NSLOT,), jnp.float32),
            jax.ShapeDtypeStruct((NTILE * 16,), jnp.int32),
        ),
        mesh=mesh1,
        scratch_types=[
            pltpu.VMEM((PP,), jnp.int32),
            pltpu.VMEM((PP,), jnp.float32),
            pltpu.VMEM((PP,), jnp.int32),
            pltpu.VMEM((PP,), jnp.int32),
            pltpu.VMEM((PP,), jnp.int32),
            pltpu.VMEM((PP // 2,), jnp.int32),
            pltpu.VMEM((PP // 2,), jnp.int32),
            pltpu.VMEM((16,), jnp.int32),
            pltpu.VMEM((NTILE * 16,), jnp.int32),
            pltpu.VMEM((16,), jnp.int32),
            pltpu.VMEM((2 * NTILE,), jnp.int32),
            pltpu.VMEM((NSLOT // NTILE,), jnp.int32),
            pltpu.VMEM((NSLOT // NTILE,), jnp.int32),
            pltpu.VMEM((NSLOT // NTILE,), jnp.float32),
        ],
        compiler_params=cp,
    )
    be, dest, tok, wvec, _ = dispatch(sel_all, w_all)

    wg16 = W_g.astype(jnp.bfloat16)
    wu16 = W_u.astype(jnp.bfloat16)
    wd16 = W_d.astype(jnp.bfloat16)

    mesh2 = plsc.VectorSubcoreMesh(
        core_axis_name="c", subcore_axis_name="s", num_cores=2)
    xg = pl.kernel(
        _gather_kernel,
        out_type=jax.ShapeDtypeStruct((NSLOT, D), jnp.float32),
        mesh=mesh2,
    )(x, tok.reshape(NSLOT // GW, GW))

    yw = pl.pallas_call(
        _ffn_body,
        grid_spec=pltpu.PrefetchScalarGridSpec(
            num_scalar_prefetch=1,
            grid=(NBLK,),
            in_specs=[
                pl.BlockSpec((BLKR, D), lambda b, be: (b, 0)),
                pl.BlockSpec((1, D, F), lambda b, be: (be[b], 0, 0)),
                pl.BlockSpec((1, D, F), lambda b, be: (be[b], 0, 0)),
                pl.BlockSpec((1, F, D), lambda b, be: (be[b], 0, 0)),
                pl.BlockSpec((BLKR, 1), lambda b, be: (b, 0)),
            ],
            out_specs=pl.BlockSpec((BLKR, D), lambda b, be: (b, 0)),
        ),
        out_shape=jax.ShapeDtypeStruct((NSLOT, D), jnp.float32),
    )(be, xg, wg16, wu16, wd16, wvec.reshape(NSLOT, 1))

    buf = pl.kernel(
        _scatter_kernel,
        out_type=jax.ShapeDtypeStruct((P2 + 8, D), jnp.float32),
        mesh=mesh2,
    )(yw, dest.reshape(NSLOT // GW, GW))

    BA = 512
    out = pl.pallas_call(
        _add_body,
        grid=(T // BA,),
        in_specs=[
            pl.BlockSpec((BA, D), lambda t: (t, 0)),
            pl.BlockSpec((BA, D), lambda t: (t + T // BA, 0)),
        ],
        out_specs=pl.BlockSpec((BA, D), lambda t: (t, 0)),
        out_shape=jax.ShapeDtypeStruct((T, D), jnp.float32),
    )(buf, buf)
    return out.reshape(orig_shape)


# speedup vs baseline: 1.2986x; 1.2986x over previous
import dataclasses
import functools

import jax
import jax.numpy as jnp
from jax import lax
from jax.experimental import pallas as pl
from jax.experimental.pallas import tpu as pltpu
from jax.experimental.pallas import tpu_sc as plsc

E = 8
D = 1024
F = 768
T = 2048
P2 = 2 * T
BLKR = 128
NBLK = P2 // BLKR + E
NSLOT = NBLK * BLKR
TRASH = P2
NBE = 48

NTILE = 16
PP = P2 // NTILE
NCH = PP // 16

NW = 32
RPW = NSLOT // NW
CH = 32
NCHG = RPW // CH


def _router_body(x_ref, wg_ref, s1_ref, s2_ref, w1_ref, w2_ref):
    x = x_ref[...].astype(jnp.bfloat16)
    gw = wg_ref[...].astype(jnp.bfloat16)
    logits = jax.lax.dot_general(
        x, gw, (((1,), (1,)), ((), ())),
        preferred_element_type=jnp.float32,
    )
    m = jnp.max(logits, axis=1, keepdims=True)
    p = jnp.exp(logits - m)
    p = p / jnp.sum(p, axis=1, keepdims=True)
    lane = jax.lax.broadcasted_iota(jnp.int32, (T, E), 1)
    m1 = jnp.max(p, axis=1, keepdims=True)
    a1 = jnp.min(jnp.where(p == m1, lane, E), axis=1, keepdims=True)
    pm = jnp.where(lane == a1, -1.0, p)
    m2 = jnp.max(pm, axis=1, keepdims=True)
    a2 = jnp.min(jnp.where(pm == m2, lane, E), axis=1, keepdims=True)
    s = m1 + m2
    s1_ref[...] = a1
    s2_ref[...] = a2
    w1_ref[...] = m1 / s
    w2_ref[...] = m2 / s


def _dispatch_body(sel_hbm, w_hbm, be_hbm, dest_hbm, wvec_hbm, lc_hbm,
                   sel_v, w_v, lrank_v, p_v, s_v0, s_v1, run_v, lc_all,
                   offs_v, be_v, zc_v, zf_v):
    wid = lax.axis_index("s")
    lane16 = jax.lax.broadcasted_iota(jnp.int32, (16,), 0)
    zeros16 = jnp.zeros((16,), jnp.int32)

    share = NSLOT // NTILE
    for j in range(share // 16):
        sl = pl.ds(j * 16, 16)
        zc_v[sl] = zeros16 + TRASH
        zf_v[sl] = jnp.zeros((16,), jnp.float32)
    base_s = wid * share
    pltpu.sync_copy(zc_v, dest_hbm.at[pl.ds(base_s, share)])
    pltpu.sync_copy(zf_v, wvec_hbm.at[pl.ds(base_s, share)])

    base_p = wid * PP
    pltpu.sync_copy(sel_hbm.at[pl.ds(base_p, PP)], sel_v)
    pltpu.sync_copy(w_hbm.at[pl.ds(base_p, PP)], w_v)
    rs = [zeros16] * E
    for i in range(NCH):
        sl = pl.ds(i * 16, 16)
        v = sel_v[sl]
        lrank = zeros16
        for e in range(E):
            msk = v == e
            c = plsc.cumsum(jnp.where(msk, 1, 0))
            cntv = plsc.all_reduce_population_count(msk)
            lrank = lrank + jnp.where(msk, rs[e] + c - 1, 0)
            rs[e] = rs[e] + cntv
        lrank_v[sl] = lrank
    run = zeros16
    for e in range(E):
        run = run + jnp.where(lane16 == e, rs[e], 0)
    run_v[...] = run
    pltpu.sync_copy(run_v, lc_hbm.at[pl.ds(wid * 16, 16)])
    plsc.subcore_barrier()

    pltpu.sync_copy(lc_hbm, lc_all)
    wid_v = zeros16 + wid
    pre = zeros16
    tot = zeros16
    for k2 in range(NTILE):
        row = lc_all[pl.ds(k2 * 16, 16)]
        tot = tot + row
        pre = pre + jnp.where(zeros16 + k2 < wid_v, row, 0)
    pe = ((tot + (BLKR - 1)) >> 7) << 7
    seg_incl = plsc.cumsum(pe)
    seg_off = seg_incl - pe
    offs_v[...] = seg_off + pre

    bstart = seg_off >> 7
    for j in range(NBE // 16):
        bvec = lane16 + 16 * j
        acc = zeros16 - 1
        for e in range(E):
            bs_e = jnp.sum(jnp.where(lane16 == e, bstart, 0))
            acc = acc + jnp.where(bvec >= zeros16 + bs_e, 1, 0)
        be_v[pl.ds(16 * j, 16)] = acc

    @pl.when(wid == 0)
    def _():
        pltpu.sync_copy(be_v, be_hbm)

    for i in range(NCH):
        sl = pl.ds(i * 16, 16)
        v = sel_v[sl]
        ge = plsc.load_gather(offs_v, [v])
        s = ge + lrank_v[sl]
        if i < NCH // 2:
            s_v0[pl.ds(i * 16, 16)] = s
        else:
            s_v1[pl.ds((i - NCH // 2) * 16, 16)] = s
        p_v[sl] = zeros16 + (base_p + i * 16) + lane16
    half = PP // 2
    pltpu.sync_copy(p_v.at[pl.ds(0, half)], dest_hbm.at[s_v0])
    pltpu.sync_copy(p_v.at[pl.ds(half, half)], dest_hbm.at[s_v1])
    pltpu.sync_copy(w_v.at[pl.ds(0, half)], wvec_hbm.at[s_v0])
    pltpu.sync_copy(w_v.at[pl.ds(half, half)], wvec_hbm.at[s_v1])


def _gather_body(x_hbm, dest_hbm, xg_hbm, didx_v, tidx_v, rv0, rv1,
                 sem0, sem1):
    wid = lax.axis_index("s") * 2 + lax.axis_index("c")
    base = wid * RPW
    pltpu.sync_copy(dest_hbm.at[pl.ds(base, RPW)], didx_v)
    for j in range(RPW // 16):
        sl = pl.ds(j * 16, 16)
        tidx_v[sl] = didx_v[sl] & (T - 1)
    bufs = (rv0, rv1)
    sems = (sem0, sem1)

    def start(c):
        return pltpu.async_copy(
            x_hbm.at[tidx_v.at[pl.ds(c * CH, CH)]], bufs[c % 2], sems[c % 2])

    h = start(0)
    for c in range(NCHG):
        nxt = start(c + 1) if c + 1 < NCHG else None
        h.wait()
        pltpu.sync_copy(bufs[c % 2], xg_hbm.at[pl.ds(base + c * CH, CH)])
        h = nxt


def _scatter_body(yw_hbm, dest_hbm, buf_hbm, dflat_v, d0, d1, d2, d3, d4,
                  rv0, rv1, semr0, semr1, semw0, semw1):
    wid = lax.axis_index("s") * 2 + lax.axis_index("c")
    base = wid * RPW
    pltpu.sync_copy(dest_hbm.at[pl.ds(base, RPW)], dflat_v)
    dc = (d0, d1, d2, d3, d4)
    for c in range(NCHG):
        for j in range(CH // 16):
            dc[c][pl.ds(j * 16, 16)] = dflat_v[pl.ds(c * CH + j * 16, 16)]
    bufs = (rv0, rv1)
    semr = (semr0, semr1)
    semw = (semw0, semw1)

    def start_read(c):
        return pltpu.async_copy(
            yw_hbm.at[pl.ds(base + c * CH, CH)], bufs[c % 2], semr[c % 2])

    hr = start_read(0)
    hw = [None, None]
    for c in range(NCHG):
        hr.wait()
        hwc = pltpu.async_copy(
            bufs[c % 2], buf_hbm.at[dc[c]], semw[c % 2])
        if c + 1 < NCHG:
            if hw[(c + 1) % 2] is not None:
                hw[(c + 1) % 2].wait()
            hr = start_read(c + 1)
        hw[c % 2] = hwc
    for k in range(2):
        if hw[k] is not None:
            hw[k].wait()


def _ffn_body(be_ref, xg_ref, wg_ref, wu_ref, wd_ref, wv_ref, yw_ref):
    xb = xg_ref[...].astype(jnp.bfloat16)
    g = jnp.dot(xb, wg_ref[0], preferred_element_type=jnp.float32)
    u = jnp.dot(xb, wu_ref[0], preferred_element_type=jnp.float32)
    h = (g * jax.nn.sigmoid(g)) * u
    y = jnp.dot(h.astype(jnp.bfloat16), wd_ref[0],
                preferred_element_type=jnp.float32)
    yw_ref[...] = y * wv_ref[...]


def _add_body(a_ref, b_ref, out_ref):
    out_ref[...] = a_ref[...] + b_ref[...]


@jax.jit
def kernel(hidden_states, W_gate, W_g, W_u, W_d):
    orig_shape = hidden_states.shape
    x = hidden_states.reshape(T, D)

    s1, s2, w1, w2 = pl.pallas_call(
        _router_body,
        out_shape=(
            jax.ShapeDtypeStruct((T, 1), jnp.int32),
            jax.ShapeDtypeStruct((T, 1), jnp.int32),
            jax.ShapeDtypeStruct((T, 1), jnp.float32),
            jax.ShapeDtypeStruct((T, 1), jnp.float32),
        ),
    )(x, W_gate)
    sel_all = jnp.concatenate([s1, s2], axis=0).reshape(P2)
    w_all = jnp.concatenate([w1, w2], axis=0).reshape(P2)

    cp = pltpu.CompilerParams()
    if "needs_layout_passes" in pltpu.CompilerParams.__dataclass_fields__:
        cp = dataclasses.replace(cp, needs_layout_passes=False)
    mesh1 = plsc.VectorSubcoreMesh(
        core_axis_name="c", subcore_axis_name="s", num_cores=1)
    be, dest, wvec, _ = pl.kernel(
        _dispatch_body,
        out_type=(
            jax.ShapeDtypeStruct((NBE,), jnp.int32),
            jax.ShapeDtypeStruct((NSLOT,), jnp.int32),
            jax.ShapeDtypeStruct((NSLOT,), jnp.float32),
            jax.ShapeDtypeStruct((NTILE * 16,), jnp.int32),
        ),
        mesh=mesh1,
        scratch_types=[
            pltpu.VMEM((PP,), jnp.int32),
            pltpu.VMEM((PP,), jnp.float32),
            pltpu.VMEM((PP,), jnp.int32),
            pltpu.VMEM((PP,), jnp.int32),
            pltpu.VMEM((PP // 2,), jnp.int32),
            pltpu.VMEM((PP // 2,), jnp.int32),
            pltpu.VMEM((16,), jnp.int32),
            pltpu.VMEM((NTILE * 16,), jnp.int32),
            pltpu.VMEM((16,), jnp.int32),
            pltpu.VMEM((NBE,), jnp.int32),
            pltpu.VMEM((NSLOT // NTILE,), jnp.int32),
            pltpu.VMEM((NSLOT // NTILE,), jnp.float32),
        ],
        compiler_params=cp,
    )(sel_all, w_all)

    wg16 = W_g.astype(jnp.bfloat16)
    wu16 = W_u.astype(jnp.bfloat16)
    wd16 = W_d.astype(jnp.bfloat16)

    mesh2 = plsc.VectorSubcoreMesh(
        core_axis_name="c", subcore_axis_name="s", num_cores=2)
    xg = pl.kernel(
        _gather_body,
        out_type=jax.ShapeDtypeStruct((NSLOT, D), jnp.float32),
        mesh=mesh2,
        scratch_types=[
            pltpu.VMEM((RPW,), jnp.int32),
            pltpu.VMEM((RPW,), jnp.int32),
            pltpu.VMEM((CH, D), jnp.float32),
            pltpu.VMEM((CH, D), jnp.float32),
            pltpu.SemaphoreType.DMA,
            pltpu.SemaphoreType.DMA,
        ],
        compiler_params=cp,
    )(x, dest)

    yw = pl.pallas_call(
        _ffn_body,
        grid_spec=pltpu.PrefetchScalarGridSpec(
            num_scalar_prefetch=1,
            grid=(NBLK,),
            in_specs=[
                pl.BlockSpec((BLKR, D), lambda b, be: (b, 0)),
                pl.BlockSpec((1, D, F), lambda b, be: (be[b], 0, 0)),
                pl.BlockSpec((1, D, F), lambda b, be: (be[b], 0, 0)),
                pl.BlockSpec((1, F, D), lambda b, be: (be[b], 0, 0)),
                pl.BlockSpec((BLKR, 1), lambda b, be: (b, 0)),
            ],
            out_specs=pl.BlockSpec((BLKR, D), lambda b, be: (b, 0)),
        ),
        out_shape=jax.ShapeDtypeStruct((NSLOT, D), jnp.float32),
    )(be, xg, wg16, wu16, wd16, wvec.reshape(NSLOT, 1))

    buf = pl.kernel(
        _scatter_body,
        out_type=jax.ShapeDtypeStruct((P2 + 8, D), jnp.float32),
        mesh=mesh2,
        scratch_types=[
            pltpu.VMEM((RPW,), jnp.int32),
            pltpu.VMEM((CH,), jnp.int32),
            pltpu.VMEM((CH,), jnp.int32),
            pltpu.VMEM((CH,), jnp.int32),
            pltpu.VMEM((CH,), jnp.int32),
            pltpu.VMEM((CH,), jnp.int32),
            pltpu.VMEM((CH, D), jnp.float32),
            pltpu.VMEM((CH, D), jnp.float32),
            pltpu.SemaphoreType.DMA,
            pltpu.SemaphoreType.DMA,
            pltpu.SemaphoreType.DMA,
            pltpu.SemaphoreType.DMA,
        ],
        compiler_params=cp,
    )(yw, dest)

    BA = 512
    out = pl.pallas_call(
        _add_body,
        grid=(T // BA,),
        in_specs=[
            pl.BlockSpec((BA, D), lambda t: (t, 0)),
            pl.BlockSpec((BA, D), lambda t: (t + T // BA, 0)),
        ],
        out_specs=pl.BlockSpec((BA, D), lambda t: (t, 0)),
        out_shape=jax.ShapeDtypeStruct((T, D), jnp.float32),
    )(buf, buf)
    return out.reshape(orig_shape)


# speedup vs baseline: 1.3574x; 1.0453x over previous
import dataclasses
import functools

import jax
import jax.numpy as jnp
from jax import lax
from jax.experimental import pallas as pl
from jax.experimental.pallas import tpu as pltpu
from jax.experimental.pallas import tpu_sc as plsc

E = 8
D = 1024
F = 768
T = 2048
P2 = 2 * T
BLKR = 128
NBLK = P2 // BLKR + E
NSLOT = NBLK * BLKR
TRASH = P2
NBE = 48

NTILE = 16
PP = P2 // NTILE
NCH = PP // 16

NW = 32
RPW = NSLOT // NW
CH = 40
NCHG = RPW // CH


def _router_body(x_ref, wg_ref, s_ref, w_ref):
    x = x_ref[...].astype(jnp.bfloat16)
    gw = wg_ref[...].astype(jnp.bfloat16)
    logits = jax.lax.dot_general(
        x, gw, (((1,), (1,)), ((), ())),
        preferred_element_type=jnp.float32,
    )
    m = jnp.max(logits, axis=1, keepdims=True)
    p = jnp.exp(logits - m)
    p = p / jnp.sum(p, axis=1, keepdims=True)
    lane = jax.lax.broadcasted_iota(jnp.int32, (T, E), 1)
    m1 = jnp.max(p, axis=1, keepdims=True)
    a1 = jnp.min(jnp.where(p == m1, lane, E), axis=1, keepdims=True)
    pm = jnp.where(lane == a1, -1.0, p)
    m2 = jnp.max(pm, axis=1, keepdims=True)
    a2 = jnp.min(jnp.where(pm == m2, lane, E), axis=1, keepdims=True)
    s = m1 + m2
    s_ref[pl.ds(0, T), :] = a1
    s_ref[pl.ds(T, T), :] = a2
    w_ref[pl.ds(0, T), :] = m1 / s
    w_ref[pl.ds(T, T), :] = m2 / s


def _dispatch_body(sel_hbm, w_hbm, be_hbm, dest_hbm, wvec_hbm, lc_hbm,
                   sel_v, w_v, lrank_v, p_v, s_v0, s_v1, run_v, lc_all,
                   offs_v, be_v, zc_v, zf_v, sz0, sz1, sl0, sl1,
                   sc0, sc1, sc2, sc3):
    wid = lax.axis_index("s")
    lane16 = jax.lax.broadcasted_iota(jnp.int32, (16,), 0)
    zeros16 = jnp.zeros((16,), jnp.int32)

    share = NSLOT // NTILE
    for j in range(share // 16):
        sl = pl.ds(j * 16, 16)
        zc_v[sl] = zeros16 + TRASH
        zf_v[sl] = jnp.zeros((16,), jnp.float32)
    base_s = wid * share
    hz0 = pltpu.async_copy(zc_v, dest_hbm.at[pl.ds(base_s, share)], sz0)
    hz1 = pltpu.async_copy(zf_v, wvec_hbm.at[pl.ds(base_s, share)], sz1)

    base_p = wid * PP
    hl0 = pltpu.async_copy(sel_hbm.at[pl.ds(base_p, PP)], sel_v, sl0)
    hl1 = pltpu.async_copy(w_hbm.at[pl.ds(base_p, PP)], w_v, sl1)
    hl0.wait()
    hl1.wait()
    rs = [zeros16] * E
    for i in range(NCH):
        sl = pl.ds(i * 16, 16)
        v = sel_v[sl]
        lrank = zeros16
        for e in range(E):
            msk = v == e
            c = plsc.cumsum(jnp.where(msk, 1, 0))
            cntv = plsc.all_reduce_population_count(msk)
            lrank = lrank + jnp.where(msk, rs[e] + c - 1, 0)
            rs[e] = rs[e] + cntv
        lrank_v[sl] = lrank
    run = zeros16
    for e in range(E):
        run = run + jnp.where(lane16 == e, rs[e], 0)
    run_v[...] = run
    pltpu.sync_copy(run_v, lc_hbm.at[pl.ds(wid * 16, 16)])
    hz0.wait()
    hz1.wait()
    plsc.subcore_barrier()

    pltpu.sync_copy(lc_hbm, lc_all)
    wid_v = zeros16 + wid
    pre = zeros16
    tot = zeros16
    for k2 in range(NTILE):
        row = lc_all[pl.ds(k2 * 16, 16)]
        tot = tot + row
        pre = pre + jnp.where(zeros16 + k2 < wid_v, row, 0)
    pe = ((tot + (BLKR - 1)) >> 7) << 7
    seg_incl = plsc.cumsum(pe)
    seg_off = seg_incl - pe
    offs_v[...] = seg_off + pre

    bstart = seg_off >> 7
    for j in range(NBE // 16):
        bvec = lane16 + 16 * j
        acc = zeros16 - 1
        for e in range(E):
            bs_e = jnp.sum(jnp.where(lane16 == e, bstart, 0))
            acc = acc + jnp.where(bvec >= zeros16 + bs_e, 1, 0)
        be_v[pl.ds(16 * j, 16)] = acc

    @pl.when(wid == 0)
    def _():
        pltpu.sync_copy(be_v, be_hbm)

    for i in range(NCH):
        sl = pl.ds(i * 16, 16)
        v = sel_v[sl]
        ge = plsc.load_gather(offs_v, [v])
        s = ge + lrank_v[sl]
        if i < NCH // 2:
            s_v0[pl.ds(i * 16, 16)] = s
        else:
            s_v1[pl.ds((i - NCH // 2) * 16, 16)] = s
        p_v[sl] = zeros16 + (base_p + i * 16) + lane16
    half = PP // 2
    h0 = pltpu.async_copy(p_v.at[pl.ds(0, half)], dest_hbm.at[s_v0], sc0)
    h1 = pltpu.async_copy(p_v.at[pl.ds(half, half)], dest_hbm.at[s_v1], sc1)
    h2 = pltpu.async_copy(w_v.at[pl.ds(0, half)], wvec_hbm.at[s_v0], sc2)
    h3 = pltpu.async_copy(w_v.at[pl.ds(half, half)], wvec_hbm.at[s_v1], sc3)
    h0.wait()
    h1.wait()
    h2.wait()
    h3.wait()


def _gather_body(x_hbm, dest_hbm, xg_hbm, didx_v, tidx_v, rv0, rv1, rv2,
                 sg0, sg1, sg2, sw0, sw1, sw2):
    wid = lax.axis_index("s") * 2 + lax.axis_index("c")
    base = wid * RPW
    pltpu.sync_copy(dest_hbm.at[pl.ds(base, RPW)], didx_v)
    for j in range(RPW // 16):
        sl = pl.ds(j * 16, 16)
        tidx_v[sl] = didx_v[sl] & (T - 1)
    bufs = (rv0, rv1, rv2)
    sg = (sg0, sg1, sg2)
    sw = (sw0, sw1, sw2)

    def g(c):
        return pltpu.async_copy(
            x_hbm.at[tidx_v.at[pl.ds(c * CH, CH)]], bufs[c % 3], sg[c % 3])

    def w(c):
        return pltpu.async_copy(
            bufs[c % 3], xg_hbm.at[pl.ds(base + c * CH, CH)], sw[c % 3])

    hg0, hg1, hg2 = g(0), g(1), g(2)
    hg0.wait()
    hw0 = w(0)
    hg1.wait()
    hw1 = w(1)
    hg2.wait()
    hw2 = w(2)
    hw0.wait()
    hg3 = g(3)
    hg3.wait()
    hw3 = w(3)
    hw1.wait()
    hw2.wait()
    hw3.wait()


def _scatter_body(yw_hbm, dest_hbm, buf_hbm, d0, d1, d2, d3,
                  rv0, rv1, rv2, si0, si1, si2, si3,
                  sr0, sr1, sr2, sw0, sw1, sw2):
    wid = lax.axis_index("s") * 2 + lax.axis_index("c")
    base = wid * RPW
    dc = (d0, d1, d2, d3)
    si = (si0, si1, si2, si3)
    hi = [pltpu.async_copy(dest_hbm.at[pl.ds(base + c * CH, CH)], dc[c],
                           si[c]) for c in range(NCHG)]
    bufs = (rv0, rv1, rv2)
    sr = (sr0, sr1, sr2)
    sw = (sw0, sw1, sw2)

    def r(c):
        return pltpu.async_copy(
            yw_hbm.at[pl.ds(base + c * CH, CH)], bufs[c % 3], sr[c % 3])

    def w(c):
        return pltpu.async_copy(bufs[c % 3], buf_hbm.at[dc[c]], sw[c % 3])

    hr0, hr1, hr2 = r(0), r(1), r(2)
    hi[0].wait()
    hr0.wait()
    hw0 = w(0)
    hi[1].wait()
    hr1.wait()
    hw1 = w(1)
    hi[2].wait()
    hr2.wait()
    hw2 = w(2)
    hw0.wait()
    hr3 = r(3)
    hi[3].wait()
    hr3.wait()
    hw3 = w(3)
    hw1.wait()
    hw2.wait()
    hw3.wait()


def _ffn_body(be_ref, xg_ref, wg_ref, wu_ref, wd_ref, wv_ref, yw_ref):
    xb = xg_ref[...].astype(jnp.bfloat16)
    g = jnp.dot(xb, wg_ref[0], preferred_element_type=jnp.float32)
    u = jnp.dot(xb, wu_ref[0], preferred_element_type=jnp.float32)
    h = (g * jax.nn.sigmoid(g)) * u
    y = jnp.dot(h.astype(jnp.bfloat16), wd_ref[0],
                preferred_element_type=jnp.float32)
    yw_ref[...] = y * wv_ref[...]


def _add_body(a_ref, b_ref, out_ref):
    out_ref[...] = a_ref[...] + b_ref[...]


@jax.jit
def kernel(hidden_states, W_gate, W_g, W_u, W_d):
    orig_shape = hidden_states.shape
    x = hidden_states.reshape(T, D)

    sel2d, w2d = pl.pallas_call(
        _router_body,
        out_shape=(
            jax.ShapeDtypeStruct((P2, 1), jnp.int32),
            jax.ShapeDtypeStruct((P2, 1), jnp.float32),
        ),
    )(x, W_gate)
    sel_all = sel2d.reshape(P2)
    w_all = w2d.reshape(P2)

    cp = pltpu.CompilerParams()
    if "needs_layout_passes" in pltpu.CompilerParams.__dataclass_fields__:
        cp = dataclasses.replace(cp, needs_layout_passes=False)
    mesh1 = plsc.VectorSubcoreMesh(
        core_axis_name="c", subcore_axis_name="s", num_cores=1)
    be, dest, wvec, _ = pl.kernel(
        _dispatch_body,
        out_type=(
            jax.ShapeDtypeStruct((NBE,), jnp.int32),
            jax.ShapeDtypeStruct((NSLOT,), jnp.int32),
            jax.ShapeDtypeStruct((NSLOT,), jnp.float32),
            jax.ShapeDtypeStruct((NTILE * 16,), jnp.int32),
        ),
        mesh=mesh1,
        scratch_types=[
            pltpu.VMEM((PP,), jnp.int32),
            pltpu.VMEM((PP,), jnp.float32),
            pltpu.VMEM((PP,), jnp.int32),
            pltpu.VMEM((PP,), jnp.int32),
            pltpu.VMEM((PP // 2,), jnp.int32),
            pltpu.VMEM((PP // 2,), jnp.int32),
            pltpu.VMEM((16,), jnp.int32),
            pltpu.VMEM((NTILE * 16,), jnp.int32),
            pltpu.VMEM((16,), jnp.int32),
            pltpu.VMEM((NBE,), jnp.int32),
            pltpu.VMEM((NSLOT // NTILE,), jnp.int32),
            pltpu.VMEM((NSLOT // NTILE,), jnp.float32),
        ] + [pltpu.SemaphoreType.DMA] * 8,
        compiler_params=cp,
    )(sel_all, w_all)

    wg16 = W_g.astype(jnp.bfloat16)
    wu16 = W_u.astype(jnp.bfloat16)
    wd16 = W_d.astype(jnp.bfloat16)

    mesh2 = plsc.VectorSubcoreMesh(
        core_axis_name="c", subcore_axis_name="s", num_cores=2)
    xg = pl.kernel(
        _gather_body,
        out_type=jax.ShapeDtypeStruct((NSLOT, D), jnp.float32),
        mesh=mesh2,
        scratch_types=[
            pltpu.VMEM((RPW,), jnp.int32),
            pltpu.VMEM((RPW,), jnp.int32),
            pltpu.VMEM((CH, D), jnp.float32),
            pltpu.VMEM((CH, D), jnp.float32),
            pltpu.VMEM((CH, D), jnp.float32),
        ] + [pltpu.SemaphoreType.DMA] * 6,
        compiler_params=cp,
    )(x, dest)

    yw = pl.pallas_call(
        _ffn_body,
        grid_spec=pltpu.PrefetchScalarGridSpec(
            num_scalar_prefetch=1,
            grid=(NBLK,),
            in_specs=[
                pl.BlockSpec((BLKR, D), lambda b, be: (b, 0)),
                pl.BlockSpec((1, D, F), lambda b, be: (be[b], 0, 0)),
                pl.BlockSpec((1, D, F), lambda b, be: (be[b], 0, 0)),
                pl.BlockSpec((1, F, D), lambda b, be: (be[b], 0, 0)),
                pl.BlockSpec((BLKR, 1), lambda b, be: (b, 0)),
            ],
            out_specs=pl.BlockSpec((BLKR, D), lambda b, be: (b, 0)),
        ),
        out_shape=jax.ShapeDtypeStruct((NSLOT, D), jnp.float32),
    )(be, xg, wg16, wu16, wd16, wvec.reshape(NSLOT, 1))

    buf = pl.kernel(
        _scatter_body,
        out_type=jax.ShapeDtypeStruct((P2 + 8, D), jnp.float32),
        mesh=mesh2,
        scratch_types=[
            pltpu.VMEM((CH,), jnp.int32),
            pltpu.VMEM((CH,), jnp.int32),
            pltpu.VMEM((CH,), jnp.int32),
            pltpu.VMEM((CH,), jnp.int32),
            pltpu.VMEM((CH, D), jnp.float32),
            pltpu.VMEM((CH, D), jnp.float32),
            pltpu.VMEM((CH, D), jnp.float32),
        ] + [pltpu.SemaphoreType.DMA] * 10,
        compiler_params=cp,
    )(yw, dest)

    BA = 512
    out = pl.pallas_call(
        _add_body,
        grid=(T // BA,),
        in_specs=[
            pl.BlockSpec((BA, D), lambda t: (t, 0)),
            pl.BlockSpec((BA, D), lambda t: (t + T // BA, 0)),
        ],
        out_specs=pl.BlockSpec((BA, D), lambda t: (t, 0)),
        out_shape=jax.ShapeDtypeStruct((T, D), jnp.float32),
    )(buf, buf)
    return out.reshape(orig_shape)


# speedup vs baseline: 2.8758x; 2.1186x over previous
import jax
import jax.numpy as jnp
from jax.experimental import pallas as pl
from jax.experimental.pallas import tpu as pltpu

E = 8
D = 1024
F = 768
T = 2048
FC = 2
FH = F // FC


def _router_body(x_ref, wg_ref, wfull_ref):
    x = x_ref[...].astype(jnp.bfloat16)
    gw = wg_ref[...].astype(jnp.bfloat16)
    logits = jax.lax.dot_general(
        x, gw, (((1,), (1,)), ((), ())),
        preferred_element_type=jnp.float32,
    )
    m = jnp.max(logits, axis=1, keepdims=True)
    p = jnp.exp(logits - m)
    p = p / jnp.sum(p, axis=1, keepdims=True)
    lane = jax.lax.broadcasted_iota(jnp.int32, (T, E), 1)
    m1 = jnp.max(p, axis=1, keepdims=True)
    a1 = jnp.min(jnp.where(p == m1, lane, E), axis=1, keepdims=True)
    pm = jnp.where(lane == a1, -1.0, p)
    m2 = jnp.max(pm, axis=1, keepdims=True)
    a2 = jnp.min(jnp.where(pm == m2, lane, E), axis=1, keepdims=True)
    one1 = lane == a1
    one2 = lane == a2
    s = m1 + m2
    wfull_ref[...] = (jnp.where(one1, m1, 0.0) + jnp.where(one2, m2, 0.0)) / s


def _ffn_body(xb_ref, wg_ref, wu_ref, wd_ref, wf_ref, out_ref):
    e = pl.program_id(0)
    f = pl.program_id(1)
    xb = xb_ref[...]
    wg = wg_ref[0].astype(jnp.bfloat16)
    wu = wu_ref[0].astype(jnp.bfloat16)
    wd = wd_ref[0].astype(jnp.bfloat16)
    g = jnp.dot(xb, wg, preferred_element_type=jnp.float32)
    u = jnp.dot(xb, wu, preferred_element_type=jnp.float32)
    h = (g * jax.nn.sigmoid(g)) * u
    y = jnp.dot(h.astype(jnp.bfloat16), wd,
                preferred_element_type=jnp.float32)
    lane = jax.lax.broadcasted_iota(jnp.int32, (T, E), 1)
    we = jnp.sum(jnp.where(lane == e, wf_ref[...], 0.0), axis=1)
    contrib = y * we[:, None]

    @pl.when((e == 0) & (f == 0))
    def _():
        out_ref[...] = contrib

    @pl.when((e > 0) | (f > 0))
    def _():
        out_ref[...] = out_ref[...] + contrib


@jax.jit
def kernel(hidden_states, W_gate, W_g, W_u, W_d):
    orig_shape = hidden_states.shape
    x = hidden_states.reshape(T, D)
    wfull = pl.pallas_call(
        _router_body,
        out_shape=jax.ShapeDtypeStruct((T, E), jnp.float32),
    )(x, W_gate)

    xb = x.astype(jnp.bfloat16)
    out = pl.pallas_call(
        _ffn_body,
        grid=(E, FC),
        in_specs=[
            pl.BlockSpec((T, D), lambda e, f: (0, 0)),
            pl.BlockSpec((1, D, FH), lambda e, f: (e, 0, f)),
            pl.BlockSpec((1, D, FH), lambda e, f: (e, 0, f)),
            pl.BlockSpec((1, FH, D), lambda e, f: (e, f, 0)),
            pl.BlockSpec((T, E), lambda e, f: (0, 0)),
        ],
        out_specs=pl.BlockSpec((T, D), lambda e, f: (0, 0)),
        out_shape=jax.ShapeDtypeStruct((T, D), jnp.float32),
    )(xb, W_g, W_u, W_d, wfull)
    return out.reshape(orig_shape)


# speedup vs baseline: 3.6456x; 1.2677x over previous
import jax
import jax.numpy as jnp
from jax.experimental import pallas as pl
from jax.experimental.pallas import tpu as pltpu

E = 8
D = 1024
F = 768
T = 2048
FC = 2
FH = F // FC


def _router_body(x_ref, wg_ref, wfull_ref):
    x = x_ref[...].astype(jnp.bfloat16)
    gw = wg_ref[...].astype(jnp.bfloat16)
    logits = jax.lax.dot_general(
        x, gw, (((1,), (1,)), ((), ())),
        preferred_element_type=jnp.float32,
    )
    m = jnp.max(logits, axis=1, keepdims=True)
    p = jnp.exp(logits - m)
    p = p / jnp.sum(p, axis=1, keepdims=True)
    lane = jax.lax.broadcasted_iota(jnp.int32, (T, E), 1)
    m1 = jnp.max(p, axis=1, keepdims=True)
    a1 = jnp.min(jnp.where(p == m1, lane, E), axis=1, keepdims=True)
    pm = jnp.where(lane == a1, -1.0, p)
    m2 = jnp.max(pm, axis=1, keepdims=True)
    a2 = jnp.min(jnp.where(pm == m2, lane, E), axis=1, keepdims=True)
    one1 = lane == a1
    one2 = lane == a2
    s = m1 + m2
    wfull_ref[...] = (jnp.where(one1, m1, 0.0) + jnp.where(one2, m2, 0.0)) / s


TCH = 512


def _ffn_body(xb_ref, wg_ref, wu_ref, wd_ref, wf_ref, out_ref):
    e = pl.program_id(0)
    wg = wg_ref[0].astype(jnp.bfloat16)
    wu = wu_ref[0].astype(jnp.bfloat16)
    wd = wd_ref[0].astype(jnp.bfloat16)
    lane = jax.lax.broadcasted_iota(jnp.int32, (TCH, E), 1)
    for tc in range(T // TCH):
        sl = pl.ds(tc * TCH, TCH)
        xb = xb_ref[sl, :]
        g = jnp.dot(xb, wg, preferred_element_type=jnp.float32)
        u = jnp.dot(xb, wu, preferred_element_type=jnp.float32)
        h = (g * jax.nn.sigmoid(g)) * u
        y = jnp.dot(h.astype(jnp.bfloat16), wd,
                    preferred_element_type=jnp.float32)
        we = jnp.sum(jnp.where(lane == e, wf_ref[sl, :], 0.0), axis=1)
        contrib = y * we[:, None]

        @pl.when(e == 0)
        def _():
            out_ref[sl, :] = contrib

        @pl.when(e > 0)
        def _():
            out_ref[sl, :] = out_ref[sl, :] + contrib


@jax.jit
def kernel(hidden_states, W_gate, W_g, W_u, W_d):
    orig_shape = hidden_states.shape
    x = hidden_states.reshape(T, D)
    wfull = pl.pallas_call(
        _router_body,
        out_shape=jax.ShapeDtypeStruct((T, E), jnp.float32),
    )(x, W_gate)

    xb = x.astype(jnp.bfloat16)
    out = pl.pallas_call(
        _ffn_body,
        grid=(E,),
        in_specs=[
            pl.BlockSpec((T, D), lambda e: (0, 0)),
            pl.BlockSpec((1, D, F), lambda e: (e, 0, 0)),
            pl.BlockSpec((1, D, F), lambda e: (e, 0, 0)),
            pl.BlockSpec((1, F, D), lambda e: (e, 0, 0)),
            pl.BlockSpec((T, E), lambda e: (0, 0)),
        ],
        out_specs=pl.BlockSpec((T, D), lambda e: (0, 0)),
        out_shape=jax.ShapeDtypeStruct((T, D), jnp.float32),
    )(xb, W_g, W_u, W_d, wfull)
    return out.reshape(orig_shape)


# speedup vs baseline: 3.8453x; 1.0548x over previous
import jax
import jax.numpy as jnp
from jax.experimental import pallas as pl
from jax.experimental.pallas import tpu as pltpu

E = 8
D = 1024
F = 768
T = 2048
FC = 2
FH = F // FC


def _router_body(x_ref, wg_ref, wfull_ref, xb_ref):
    x = x_ref[...].astype(jnp.bfloat16)
    xb_ref[...] = x
    gw = wg_ref[...].astype(jnp.bfloat16)
    logits = jax.lax.dot_general(
        x, gw, (((1,), (1,)), ((), ())),
        preferred_element_type=jnp.float32,
    )
    m = jnp.max(logits, axis=1, keepdims=True)
    p = jnp.exp(logits - m)
    p = p / jnp.sum(p, axis=1, keepdims=True)
    lane = jax.lax.broadcasted_iota(jnp.int32, (T, E), 1)
    m1 = jnp.max(p, axis=1, keepdims=True)
    a1 = jnp.min(jnp.where(p == m1, lane, E), axis=1, keepdims=True)
    pm = jnp.where(lane == a1, -1.0, p)
    m2 = jnp.max(pm, axis=1, keepdims=True)
    a2 = jnp.min(jnp.where(pm == m2, lane, E), axis=1, keepdims=True)
    one1 = lane == a1
    one2 = lane == a2
    s = m1 + m2
    wfull_ref[...] = (jnp.where(one1, m1, 0.0) + jnp.where(one2, m2, 0.0)) / s


TCH = 512


def _ffn_body(xb_ref, wg_ref, wu_ref, wd_ref, wf_ref, out_ref):
    e = pl.program_id(0)
    wg = wg_ref[0].astype(jnp.bfloat16)
    wu = wu_ref[0].astype(jnp.bfloat16)
    wd = wd_ref[0].astype(jnp.bfloat16)
    lane = jax.lax.broadcasted_iota(jnp.int32, (TCH, E), 1)
    for tc in range(T // TCH):
        sl = pl.ds(tc * TCH, TCH)
        xb = xb_ref[sl, :]
        g = jnp.dot(xb, wg, preferred_element_type=jnp.float32)
        u = jnp.dot(xb, wu, preferred_element_type=jnp.float32)
        h = (g * jax.nn.sigmoid(g)) * u
        y = jnp.dot(h.astype(jnp.bfloat16), wd,
                    preferred_element_type=jnp.float32)
        we = jnp.sum(jnp.where(lane == e, wf_ref[sl, :], 0.0), axis=1)
        contrib = y * we[:, None]

        @pl.when(e == 0)
        def _():
            out_ref[sl, :] = contrib

        @pl.when(e > 0)
        def _():
            out_ref[sl, :] = out_ref[sl, :] + contrib


@jax.jit
def kernel(hidden_states, W_gate, W_g, W_u, W_d):
    orig_shape = hidden_states.shape
    x = hidden_states.reshape(T, D)
    wfull, xb = pl.pallas_call(
        _router_body,
        out_shape=(
            jax.ShapeDtypeStruct((T, E), jnp.float32),
            jax.ShapeDtypeStruct((T, D), jnp.bfloat16),
        ),
    )(x, W_gate)

    out = pl.pallas_call(
        _ffn_body,
        grid=(E,),
        in_specs=[
            pl.BlockSpec((T, D), lambda e: (0, 0)),
            pl.BlockSpec((1, D, F), lambda e: (e, 0, 0)),
            pl.BlockSpec((1, D, F), lambda e: (e, 0, 0)),
            pl.BlockSpec((1, F, D), lambda e: (e, 0, 0)),
            pl.BlockSpec((T, E), lambda e: (0, 0)),
        ],
        out_specs=pl.BlockSpec((T, D), lambda e: (0, 0)),
        out_shape=jax.ShapeDtypeStruct((T, D), jnp.float32),
    )(xb, W_g, W_u, W_d, wfull)
    return out.reshape(orig_shape)
